# baseline (speedup 1.0000x reference)
import functools

import jax
import jax.numpy as jnp
from jax import lax
from jax.experimental import pallas as pl
from jax.experimental.pallas import tpu as pltpu

_L = 16
_C = 128
_G = 8
_NQ = 8


def _phase1(B, K, ids_smem, t_any, d_any, ids_ref, u_ref, out_ref,
            tsm, dsm, state, rvec, sems):
    for b in range(B):
        state[b, 0] = K

    def chunk_copies(b, j):
        off = pl.multiple_of((ids_smem[b, j] // _C) * _C, _C)
        return (pltpu.make_async_copy(t_any.at[b, j, pl.ds(off, _C)],
                                      tsm.at[b], sems.at[0]),
                pltpu.make_async_copy(d_any.at[b, j, pl.ds(off, _C)],
                                      dsm.at[b], sems.at[1]))

    for j in range(K):
        for b in range(B):
            @pl.when(state[b, 0] == K)
            def _(b=b, j=j):
                ct, cd = chunk_copies(b, j)
                ct.start()
                cd.start()
        for b in range(B):
            @pl.when(state[b, 0] == K)
            def _(b=b, j=j):
                ct, cd = chunk_copies(b, j)
                ct.wait()
                cd.wait()
                idm = ids_smem[b, j] % _C
                rej = (tsm[b, idm] / dsm[b, idm]) < u_ref[b, j]
                state[b, 0] = jnp.where(rej, j, K)

    cp = pltpu.make_async_copy(state, rvec, sems.at[0])
    cp.start()
    cp.wait()

    ridx = rvec[...]
    lane = lax.broadcasted_iota(jnp.int32, (B, _L), 1)
    outrow = jnp.where(lane < ridx, ids_ref[...], -1)
    out_ref[...] = jnp.where(lane == _L - 1, ridx, outrow)


def _phase2(B, K, V, ridx_smem, t_any, d_any, g_ref, rcol_ref, base_ref,
            o_ref, tb, db, sems):
    i = pl.program_id(0)
    n = B // _G

    def transfers(step, slot):
        cps = []
        for s in range(_G):
            bb = step * _G + s
            r = ridx_smem[bb]
            rd = jnp.minimum(r, K - 1)
            cps.append(pltpu.make_async_copy(
                t_any.at[bb, r], tb.at[slot, s], sems.at[0, slot]))
            cps.append(pltpu.make_async_copy(
                d_any.at[bb, rd], db.at[slot, s], sems.at[1, slot]))
        return cps

    @pl.when(i == 0)
    def _():
        for c in transfers(0, 0):
            c.start()

    @pl.when(i + 1 < n)
    def _():
        for c in transfers(i + 1, (i + 1) % 2):
            c.start()

    for c in transfers(i, i % 2):
        c.wait()

    r_col = rcol_ref[...]
    t = tb[i % 2]
    d = jnp.where(r_col < K, db[i % 2], 0.0)
    raw = jnp.maximum(t - d, 0.0)
    y = jnp.log(jnp.maximum(raw, 1e-20)) + g_ref[...]
    m = jnp.max(y, axis=1, keepdims=True)
    lane = lax.broadcasted_iota(jnp.int32, (_G, V), 1)
    tok = jnp.min(jnp.where(y == m, lane, jnp.int32(2**31 - 1)),
                  axis=1, keepdims=True)
    j = lax.broadcasted_iota(jnp.int32, (_G, _L), 1)
    o_ref[...] = jnp.where(j == r_col, tok, base_ref[...])


def kernel(target_probs, draft_probs, draft_token_ids):
    B, K, V = draft_probs.shape
    dtype = jnp.float32

    rkey = jax.random.key(42)
    u = jax.random.uniform(rkey, (B, K), dtype=dtype)
    skey = jax.random.fold_in(rkey, 1)
    g = jax.random.gumbel(skey, (B, V), dtype=dtype)

    ids_pad = jnp.pad(draft_token_ids, ((0, 0), (0, _L - K)))

    out2 = pl.pallas_call(
        functools.partial(_phase1, B, K),
        grid_spec=pltpu.PrefetchScalarGridSpec(
            num_scalar_prefetch=1,
            grid=(1,),
            in_specs=[
                pl.BlockSpec(memory_space=pltpu.HBM),
                pl.BlockSpec(memory_space=pltpu.HBM),
                pl.BlockSpec((B, _L), lambda i, ids: (0, 0)),
                pl.BlockSpec(memory_space=pltpu.SMEM),
            ],
            out_specs=pl.BlockSpec((B, _L), lambda i, ids: (0, 0)),
            scratch_shapes=[
                pltpu.SMEM((B, _C), dtype),
                pltpu.SMEM((B, _C), dtype),
                pltpu.SMEM((B, 1), jnp.int32),
                pltpu.VMEM((B, 1), jnp.int32),
                pltpu.SemaphoreType.DMA((2,)),
            ],
        ),
        out_shape=jax.ShapeDtypeStruct((B, _L), jnp.int32),
    )(draft_token_ids, target_probs, draft_probs, ids_pad, u)

    ridx = out2[:, _L - 1]
    rcol = ridx.reshape(B, 1)

    res = pl.pallas_call(
        functools.partial(_phase2, B, K, V),
        grid_spec=pltpu.PrefetchScalarGridSpec(
            num_scalar_prefetch=1,
            grid=(B // _G,),
            in_specs=[
                pl.BlockSpec(memory_space=pltpu.HBM),
                pl.BlockSpec(memory_space=pltpu.HBM),
                pl.BlockSpec((_G, V), lambda i, rr: (i, 0)),
                pl.BlockSpec((_G, 1), lambda i, rr: (i, 0)),
                pl.BlockSpec((_G, _L), lambda i, rr: (i, 0)),
            ],
            out_specs=pl.BlockSpec((_G, _L), lambda i, rr: (i, 0)),
            scratch_shapes=[
                pltpu.VMEM((2, _G, V), dtype),
                pltpu.VMEM((2, _G, V), dtype),
                pltpu.SemaphoreType.DMA((2, 2)),
            ],
        ),
        out_shape=jax.ShapeDtypeStruct((B, _L), jnp.int32),
    )(ridx, target_probs, draft_probs, g, rcol, out2)

    return res[:, :K + 1]


# speedup vs baseline: 1.0280x; 1.0280x over previous
import functools

import jax
import jax.numpy as jnp
from jax import lax
from jax.experimental import pallas as pl
from jax.experimental.pallas import tpu as pltpu

_L = 16
_C = 128
_G = 8
_NQ = 8


def _phase1(B, K, ids_smem, t_any, d_any, ids_ref, u_ref, out_ref,
            tsm, dsm, state, rvec, sems):
    for b in range(B):
        state[b, 0] = K

    def chunk_copies(b, j, h):
        off = pl.multiple_of((ids_smem[b, j] // _C) * _C, _C)
        return (pltpu.make_async_copy(t_any.at[b, j, pl.ds(off, _C)],
                                      tsm.at[b, h], sems.at[0]),
                pltpu.make_async_copy(d_any.at[b, j, pl.ds(off, _C)],
                                      dsm.at[b, h], sems.at[1]))

    for j in range(0, K, 2):
        for b in range(B):
            @pl.when(state[b, 0] == K)
            def _(b=b, j=j):
                for h in range(2):
                    ct, cd = chunk_copies(b, j + h, h)
                    ct.start()
                    cd.start()
        for b in range(B):
            @pl.when(state[b, 0] == K)
            def _(b=b, j=j):
                for h in range(2):
                    ct, cd = chunk_copies(b, j + h, h)
                    ct.wait()
                    cd.wait()
                idm0 = ids_smem[b, j] % _C
                rej0 = (tsm[b, 0, idm0] / dsm[b, 0, idm0]) < u_ref[b, j]
                s1 = jnp.where(rej0, j, K)
                idm1 = ids_smem[b, j + 1] % _C
                rej1 = (tsm[b, 1, idm1] / dsm[b, 1, idm1]) < u_ref[b, j + 1]
                state[b, 0] = jnp.where((s1 == K) & rej1, j + 1, s1)

    cp = pltpu.make_async_copy(state, rvec, sems.at[0])
    cp.start()
    cp.wait()

    ridx = rvec[...]
    lane = lax.broadcasted_iota(jnp.int32, (B, _L), 1)
    outrow = jnp.where(lane < ridx, ids_ref[...], -1)
    out_ref[...] = jnp.where(lane == _L - 1, ridx, outrow)


def _phase2(B, K, V, ridx_smem, t_any, d_any, g_ref, rcol_ref, base_ref,
            o_ref, tb, db, sems):
    i = pl.program_id(0)
    n = B // _G

    def transfers(step, slot):
        cps = []
        for s in range(_G):
            bb = step * _G + s
            r = ridx_smem[bb]
            rd = jnp.minimum(r, K - 1)
            cps.append(pltpu.make_async_copy(
                t_any.at[bb, r], tb.at[slot, s], sems.at[0, slot]))
            cps.append(pltpu.make_async_copy(
                d_any.at[bb, rd], db.at[slot, s], sems.at[1, slot]))
        return cps

    @pl.when(i == 0)
    def _():
        for c in transfers(0, 0):
            c.start()

    @pl.when(i + 1 < n)
    def _():
        for c in transfers(i + 1, (i + 1) % 2):
            c.start()

    for c in transfers(i, i % 2):
        c.wait()

    r_col = rcol_ref[...]
    t = tb[i % 2]
    d = jnp.where(r_col < K, db[i % 2], 0.0)
    raw = jnp.maximum(t - d, 0.0)
    y = jnp.log(jnp.maximum(raw, 1e-20)) + g_ref[...]
    m = jnp.max(y, axis=1, keepdims=True)
    lane = lax.broadcasted_iota(jnp.int32, (_G, V), 1)
    tok = jnp.min(jnp.where(y == m, lane, jnp.int32(2**31 - 1)),
                  axis=1, keepdims=True)
    j = lax.broadcasted_iota(jnp.int32, (_G, _L), 1)
    o_ref[...] = jnp.where(j == r_col, tok, base_ref[...])


def kernel(target_probs, draft_probs, draft_token_ids):
    B, K, V = draft_probs.shape
    dtype = jnp.float32

    rkey = jax.random.key(42)
    u = jax.random.uniform(rkey, (B, K), dtype=dtype)
    skey = jax.random.fold_in(rkey, 1)
    g = jax.random.gumbel(skey, (B, V), dtype=dtype)

    ids_pad = jnp.pad(draft_token_ids, ((0, 0), (0, _L - K)))

    out2 = pl.pallas_call(
        functools.partial(_phase1, B, K),
        grid_spec=pltpu.PrefetchScalarGridSpec(
            num_scalar_prefetch=1,
            grid=(1,),
            in_specs=[
                pl.BlockSpec(memory_space=pltpu.HBM),
                pl.BlockSpec(memory_space=pltpu.HBM),
                pl.BlockSpec((B, _L), lambda i, ids: (0, 0)),
                pl.BlockSpec(memory_space=pltpu.SMEM),
            ],
            out_specs=pl.BlockSpec((B, _L), lambda i, ids: (0, 0)),
            scratch_shapes=[
                pltpu.SMEM((B, 2, _C), dtype),
                pltpu.SMEM((B, 2, _C), dtype),
                pltpu.SMEM((B, 1), jnp.int32),
                pltpu.VMEM((B, 1), jnp.int32),
                pltpu.SemaphoreType.DMA((2,)),
            ],
        ),
        out_shape=jax.ShapeDtypeStruct((B, _L), jnp.int32),
    )(draft_token_ids, target_probs, draft_probs, ids_pad, u)

    ridx = out2[:, _L - 1]
    rcol = ridx.reshape(B, 1)

    res = pl.pallas_call(
        functools.partial(_phase2, B, K, V),
        grid_spec=pltpu.PrefetchScalarGridSpec(
            num_scalar_prefetch=1,
            grid=(B // _G,),
            in_specs=[
                pl.BlockSpec(memory_space=pltpu.HBM),
                pl.BlockSpec(memory_space=pltpu.HBM),
                pl.BlockSpec((_G, V), lambda i, rr: (i, 0)),
                pl.BlockSpec((_G, 1), lambda i, rr: (i, 0)),
                pl.BlockSpec((_G, _L), lambda i, rr: (i, 0)),
            ],
            out_specs=pl.BlockSpec((_G, _L), lambda i, rr: (i, 0)),
            scratch_shapes=[
                pltpu.VMEM((2, _G, V), dtype),
                pltpu.VMEM((2, _G, V), dtype),
                pltpu.SemaphoreType.DMA((2, 2)),
            ],
        ),
        out_shape=jax.ShapeDtypeStruct((B, _L), jnp.int32),
    )(ridx, target_probs, draft_probs, g, rcol, out2)

    return res[:, :K + 1]


# speedup vs baseline: 1.0661x; 1.0370x over previous
import functools

import jax
import jax.numpy as jnp
from jax import lax
from jax.experimental import pallas as pl
from jax.experimental.pallas import tpu as pltpu

_L = 16
_C = 128
_G = 16
_NQ = 8


def _phase1(B, K, ids_smem, t_any, d_any, ids_ref, u_ref, idsm_ref, out_ref,
            tv, dv, sem):
    copies = []
    for b in range(B):
        for k in range(K):
            off = pl.multiple_of((ids_smem[b, k] // _C) * _C, _C)
            copies.append(pltpu.make_async_copy(
                t_any.at[b, k, pl.ds(off, _C)], tv.at[b, k],
                sem.at[(2 * (b * K + k)) % _NQ]))
            copies.append(pltpu.make_async_copy(
                d_any.at[b, k, pl.ds(off, _C)], dv.at[b, k],
                sem.at[(2 * (b * K + k) + 1) % _NQ]))
    for c in copies:
        c.start()
    for c in copies:
        c.wait()

    sub = lax.broadcasted_iota(jnp.int32, (B, K, _C), 2)
    mask = sub == idsm_ref[...]
    val_t = jnp.sum(jnp.where(mask, tv[...], 0.0), axis=2)
    val_d = jnp.sum(jnp.where(mask, dv[...], 0.0), axis=2)
    scores = val_t / val_d
    lane8 = lax.broadcasted_iota(jnp.int32, (B, K), 1)
    rejected = scores < u_ref[...]
    ridx = jnp.min(jnp.where(rejected, lane8, K), axis=1, keepdims=True)
    lane = lax.broadcasted_iota(jnp.int32, (B, _L), 1)
    outrow = jnp.where(lane < ridx, ids_ref[...], -1)
    out_ref[...] = jnp.where(lane == _L - 1, ridx, outrow)


def _phase2(B, K, V, ridx_smem, t_any, d_any, g_ref, rcol_ref, base_ref,
            o_ref, tb, db, sems):
    i = pl.program_id(0)
    n = B // _G

    def transfers(step, slot):
        cps = []
        for s in range(_G):
            bb = step * _G + s
            r = ridx_smem[bb]
            rd = jnp.minimum(r, K - 1)
            cps.append(pltpu.make_async_copy(
                t_any.at[bb, r], tb.at[slot, s], sems.at[0, slot]))
            cps.append(pltpu.make_async_copy(
                d_any.at[bb, rd], db.at[slot, s], sems.at[1, slot]))
        return cps

    @pl.when(i == 0)
    def _():
        for c in transfers(0, 0):
            c.start()

    @pl.when(i + 1 < n)
    def _():
        for c in transfers(i + 1, (i + 1) % 2):
            c.start()

    for c in transfers(i, i % 2):
        c.wait()

    r_col = rcol_ref[...]
    t = tb[i % 2]
    d = jnp.where(r_col < K, db[i % 2], 0.0)
    raw = jnp.maximum(t - d, 0.0)
    y = jnp.log(jnp.maximum(raw, 1e-20)) + g_ref[...]
    m = jnp.max(y, axis=1, keepdims=True)
    lane = lax.broadcasted_iota(jnp.int32, (_G, V), 1)
    tok = jnp.min(jnp.where(y == m, lane, jnp.int32(2**31 - 1)),
                  axis=1, keepdims=True)
    j = lax.broadcasted_iota(jnp.int32, (_G, _L), 1)
    o_ref[...] = jnp.where(j == r_col, tok, base_ref[...])


def kernel(target_probs, draft_probs, draft_token_ids):
    B, K, V = draft_probs.shape
    dtype = jnp.float32

    rkey = jax.random.key(42)
    u = jax.random.uniform(rkey, (B, K), dtype=dtype)
    skey = jax.random.fold_in(rkey, 1)
    g = jax.random.gumbel(skey, (B, V), dtype=dtype)

    ids_pad = jnp.pad(draft_token_ids, ((0, 0), (0, _L - K)))
    idsm = jnp.broadcast_to((draft_token_ids % _C)[:, :, None], (B, K, _C))

    out2 = pl.pallas_call(
        functools.partial(_phase1, B, K),
        grid_spec=pltpu.PrefetchScalarGridSpec(
            num_scalar_prefetch=1,
            grid=(1,),
            in_specs=[
                pl.BlockSpec(memory_space=pltpu.HBM),
                pl.BlockSpec(memory_space=pltpu.HBM),
                pl.BlockSpec((B, _L), lambda i, ids: (0, 0)),
                pl.BlockSpec((B, K), lambda i, ids: (0, 0)),
                pl.BlockSpec((B, K, _C), lambda i, ids: (0, 0, 0)),
            ],
            out_specs=pl.BlockSpec((B, _L), lambda i, ids: (0, 0)),
            scratch_shapes=[
                pltpu.VMEM((B, K, _C), dtype),
                pltpu.VMEM((B, K, _C), dtype),
                pltpu.SemaphoreType.DMA((_NQ,)),
            ],
        ),
        out_shape=jax.ShapeDtypeStruct((B, _L), jnp.int32),
    )(draft_token_ids, target_probs, draft_probs, ids_pad, u, idsm)

    ridx = out2[:, _L - 1]
    rcol = ridx.reshape(B, 1)

    res = pl.pallas_call(
        functools.partial(_phase2, B, K, V),
        grid_spec=pltpu.PrefetchScalarGridSpec(
            num_scalar_prefetch=1,
            grid=(B // _G,),
            in_specs=[
                pl.BlockSpec(memory_space=pltpu.HBM),
                pl.BlockSpec(memory_space=pltpu.HBM),
                pl.BlockSpec((_G, V), lambda i, rr: (i, 0)),
                pl.BlockSpec((_G, 1), lambda i, rr: (i, 0)),
                pl.BlockSpec((_G, _L), lambda i, rr: (i, 0)),
            ],
            out_specs=pl.BlockSpec((_G, _L), lambda i, rr: (i, 0)),
            scratch_shapes=[
                pltpu.VMEM((2, _G, V), dtype),
                pltpu.VMEM((2, _G, V), dtype),
                pltpu.SemaphoreType.DMA((2, 2)),
            ],
        ),
        out_shape=jax.ShapeDtypeStruct((B, _L), jnp.int32),
    )(ridx, target_probs, draft_probs, g, rcol, out2)

    return res[:, :K + 1]
